# Initial kernel scaffold; baseline (speedup 1.0000x reference)
import jax
import jax.numpy as jnp
from jax.experimental import pallas as pl


def kernel(x, edge_index, edge_attr, elem_emb, chir_emb, bond_type_emb, bond_dir_emb, W1, b1, W2, b2, gammas, betas):
    raise NotImplementedError("write your pallas kernel here")


# speedup vs baseline: 1.0974x; 1.0974x over previous
import functools

import jax
import jax.numpy as jnp
from jax import lax
from jax.experimental import pallas as pl
from jax.experimental.pallas import tpu as pltpu
from jax.experimental.pallas import tpu_sc as plsc

N_NODES = 10000
N_EDGES = 320000
D = 128
DH = 64

NC = 2
NS = 16
NW = NC * NS

B = 128
NP = 12288
EP = 327680
EB = EP // NS // B
NB = NP // NS // B
ROWS_PER_TILE = NP // NS

_MESH = plsc.VectorSubcoreMesh(core_axis_name="c", subcore_axis_name="s",
                               num_cores=NC, num_subcores=NS)
_SC_PARAMS = pltpu.CompilerParams(use_tc_tiling_on_sc=False)


def _zero_msg(buf):
    def row(r, carry):
        for cc in range(DH // 16):
            buf[r, pl.ds(cc * 16, 16)] = jnp.zeros((16,), jnp.float32)
        return carry
    lax.fori_loop(0, B, row, 0)


@functools.partial(
    pl.kernel,
    out_type=jax.ShapeDtypeStruct((NC * NP, DH), jnp.float32),
    mesh=_MESH,
    scratch_types=[
        pltpu.VMEM((NB, B), jnp.int32),
        pltpu.VMEM((NB, B), jnp.int32),
        pltpu.VMEM((B, DH), jnp.float32),
        pltpu.SemaphoreType.DMA,
    ],
    compiler_params=_SC_PARAMS,
)
def _node_embed(x0_hbm, x1_hbm, elem_hbm, chir_hbm, out_hbm, x0v, x1v, buf, sem):
    c = lax.axis_index("c")
    s = lax.axis_index("s")
    pltpu.sync_copy(x0_hbm.at[s], x0v)
    pltpu.sync_copy(x1_hbm.at[s], x1v)

    def offrow(r, carry):
        for cc in range(B // 16):
            sl = pl.ds(cc * 16, 16)
            x0v[r, sl] = x0v[r, sl] + c * 118
            x1v[r, sl] = x1v[r, sl] + c * 4
        return carry
    lax.fori_loop(0, NB, offrow, 0)

    for b in range(NB):
        pltpu.async_copy(elem_hbm.at[x0v.at[b]], buf, sem).wait()
        pltpu.async_copy(chir_hbm.at[x1v.at[b]], buf, sem, add=True).wait()
        pltpu.sync_copy(buf, out_hbm.at[pl.ds(c * NP + s * (NB * B) + b * B, B)])


@functools.partial(
    pl.kernel,
    out_type=jax.ShapeDtypeStruct((NC, NP, DH), jnp.float32),
    mesh=_MESH,
    scratch_types=[
        pltpu.VMEM((EB // 2, B), jnp.int32),
        pltpu.VMEM((EB // 2, B), jnp.int32),
        pltpu.VMEM((EB // 2, B), jnp.int32),
        pltpu.VMEM((EB // 2, B), jnp.int32),
        pltpu.VMEM((B, DH), jnp.float32),
        pltpu.VMEM_SHARED((NP, DH), jnp.float32),
        pltpu.SemaphoreType.DMA,
    ],
    compiler_params=_SC_PARAMS,
)
def _edge_agg(h_hbm, src_hbm, dst_hbm, ea0_hbm, ea1_hbm, fused_hbm, out_hbm,
              srcv, dstv, ecv, tmpv, msg, agg_s, sem):
    c = lax.axis_index("c")
    s = lax.axis_index("s")
    hb = EB // 2

    _zero_msg(msg)
    for k in range(ROWS_PER_TILE // B):
        pltpu.sync_copy(msg, agg_s.at[pl.ds(s * ROWS_PER_TILE + k * B, B)])

    plsc.subcore_barrier()

    def phase(ph, carry0):
        pltpu.sync_copy(src_hbm.at[s, pl.ds(ph * hb, hb)], srcv)
        pltpu.sync_copy(dst_hbm.at[s, pl.ds(ph * hb, hb)], dstv)
        pltpu.sync_copy(ea0_hbm.at[s, pl.ds(ph * hb, hb)], ecv)
        pltpu.sync_copy(ea1_hbm.at[s, pl.ds(ph * hb, hb)], tmpv)

        def idxrow(r, carry):
            for cc in range(B // 16):
                sl = pl.ds(cc * 16, 16)
                srcv[r, sl] = srcv[r, sl] + c * NP
                ecv[r, sl] = ecv[r, sl] * 3 + tmpv[r, sl] + c * 12
            return carry
        lax.fori_loop(0, hb, idxrow, 0)

        def batch(b, carry):
            pltpu.async_copy(fused_hbm.at[ecv.at[b]], msg, sem).wait()
            pltpu.async_copy(h_hbm.at[srcv.at[b]], msg, sem, add=True).wait()

            def rrow(r, carry2):
                for cc in range(DH // 16):
                    sl = pl.ds(cc * 16, 16)
                    msg[r, sl] = jnp.maximum(msg[r, sl], 0.0)
                return carry2
            lax.fori_loop(0, B, rrow, 0)

            pltpu.sync_copy(msg, agg_s.at[dstv.at[b]], add=True)
            return carry
        lax.fori_loop(0, hb, batch, 0)
        return carry0
    lax.fori_loop(0, 2, phase, 0)

    plsc.subcore_barrier()
    pltpu.sync_copy(agg_s.at[pl.ds(s * ROWS_PER_TILE, ROWS_PER_TILE)],
                    out_hbm.at[c, pl.ds(s * ROWS_PER_TILE, ROWS_PER_TILE)])


def _hp_matmul(a, w):
    ah = a.astype(jnp.bfloat16)
    al = (a - ah.astype(jnp.float32)).astype(jnp.bfloat16)
    wh = w.astype(jnp.bfloat16)
    wl = (w - wh.astype(jnp.float32)).astype(jnp.bfloat16)
    dn = (((1,), (0,)), ((), ()))
    mm = functools.partial(lax.dot_general, dimension_numbers=dn,
                           preferred_element_type=jnp.float32)
    return mm(ah, wh) + (mm(ah, wl) + mm(al, wh))


def _mlp_bn_body(apply_relu, h_ref, a_ref, w1_ref, b1_ref, w2_ref, b2_ref,
                 g_ref, be_ref, o_ref):
    z = jnp.concatenate([h_ref[0] + a_ref[0], h_ref[1] + a_ref[1]], axis=1)
    t = _hp_matmul(z, w1_ref[...]) + b1_ref[...]
    t = jnp.maximum(t, 0.0)
    z2 = _hp_matmul(t, w2_ref[...]) + b2_ref[...]
    mask = lax.broadcasted_iota(jnp.int32, (NP, 1), 0) < N_NODES
    zm = jnp.where(mask, z2, 0.0)
    inv_n = 1.0 / N_NODES
    mu = jnp.sum(zm, axis=0, keepdims=True) * inv_n
    zc = jnp.where(mask, z2 - mu, 0.0)
    var = jnp.sum(zc * zc, axis=0, keepdims=True) * inv_n
    hn = (z2 - mu) * lax.rsqrt(var + 1e-5) * g_ref[...] + be_ref[...]
    if apply_relu:
        hn = jnp.maximum(hn, 0.0)
    o_ref[0] = hn[:, :DH]
    o_ref[1] = hn[:, DH:]


def _mlp_bn(h2, aggs, W1, b1, W2, b2, gamma, beta, apply_relu):
    return pl.pallas_call(
        functools.partial(_mlp_bn_body, apply_relu),
        out_shape=jax.ShapeDtypeStruct((NC, NP, DH), jnp.float32),
    )(h2, aggs, W1, b1.reshape(1, 2 * D), W2, b2.reshape(1, D),
      gamma.reshape(1, D), beta.reshape(1, D))


def _col_split(t):
    r = t.shape[0]
    return jnp.stack([t[:, :DH], t[:, DH:]]).reshape(2 * r, DH)


def kernel(x, edge_index, edge_attr, elem_emb, chir_emb, bond_type_emb,
           bond_dir_emb, W1, b1, W2, b2, gammas, betas):
    x0 = jnp.pad(x[:, 0], (0, NP - N_NODES)).reshape(NS, NB, B)
    x1 = jnp.pad(x[:, 1], (0, NP - N_NODES)).reshape(NS, NB, B)
    src = jnp.pad(edge_index[0], (0, EP - N_EDGES)).reshape(NS, EB, B)
    dst = jnp.pad(edge_index[1], (0, EP - N_EDGES),
                  constant_values=NP - 1).reshape(NS, EB, B)
    ea0 = jnp.pad(edge_attr[:, 0], (0, EP - N_EDGES)).reshape(NS, EB, B)
    ea1 = jnp.pad(edge_attr[:, 1], (0, EP - N_EDGES)).reshape(NS, EB, B)
    fused = _col_split(
        (bond_type_emb[:, None, :] + bond_dir_emb[None, :, :]).reshape(12, D))
    elem2 = _col_split(elem_emb)
    chir2 = _col_split(chir_emb)

    h = _node_embed(x0, x1, elem2, chir2)
    for layer in range(3):
        aggs = _edge_agg(h, src, dst, ea0, ea1, fused)
        hf = jnp.concatenate([h[:NP], h[NP:]], axis=1)
        af = jnp.concatenate([aggs[0], aggs[1]], axis=1)
        z = hf + af
        z2 = jax.nn.relu(z @ W1 + b1) @ W2 + b2
        mask = (jnp.arange(NP) < N_NODES)[:, None]
        zm = jnp.where(mask, z2, 0.0)
        mu = jnp.sum(zm, 0) / N_NODES
        zc = jnp.where(mask, z2 - mu, 0.0)
        var = jnp.sum(zc * zc, 0) / N_NODES
        hn = (z2 - mu) * jax.lax.rsqrt(var + 1e-5) * gammas[layer] + betas[layer]
        if layer < 2:
            hn = jax.nn.relu(hn)
        h = jnp.concatenate([hn[:, :DH], hn[:, DH:]], axis=0)
    out = h.reshape(NC, NP, DH)
    return jnp.concatenate([out[0], out[1]], axis=1)[:N_NODES]


# speedup vs baseline: 1.1677x; 1.0640x over previous
import functools

import jax
import jax.numpy as jnp
from jax import lax
from jax.experimental import pallas as pl
from jax.experimental.pallas import tpu as pltpu
from jax.experimental.pallas import tpu_sc as plsc

N_NODES = 10000
N_EDGES = 320000
D = 128
DH = 64

NC = 2
NS = 16
NW = NC * NS

B = 128
NP = 12288
EP = 327680
EB = EP // NS // B
NB = NP // NS // B
ROWS_PER_TILE = NP // NS

_MESH = plsc.VectorSubcoreMesh(core_axis_name="c", subcore_axis_name="s",
                               num_cores=NC, num_subcores=NS)
_SC_PARAMS = pltpu.CompilerParams(use_tc_tiling_on_sc=False)


def _zero_msg(buf):
    def row(r, carry):
        for cc in range(DH // 16):
            buf[r, pl.ds(cc * 16, 16)] = jnp.zeros((16,), jnp.float32)
        return carry
    lax.fori_loop(0, B, row, 0)


@functools.partial(
    pl.kernel,
    out_type=jax.ShapeDtypeStruct((NC * NP, DH), jnp.float32),
    mesh=_MESH,
    scratch_types=[
        pltpu.VMEM((NB, B), jnp.int32),
        pltpu.VMEM((NB, B), jnp.int32),
        pltpu.VMEM((B, DH), jnp.float32),
        pltpu.SemaphoreType.DMA,
    ],
    compiler_params=_SC_PARAMS,
)
def _node_embed(x0_hbm, x1_hbm, elem_hbm, chir_hbm, out_hbm, x0v, x1v, buf, sem):
    c = lax.axis_index("c")
    s = lax.axis_index("s")
    pltpu.sync_copy(x0_hbm.at[s], x0v)
    pltpu.sync_copy(x1_hbm.at[s], x1v)

    def offrow(r, carry):
        for cc in range(B // 16):
            sl = pl.ds(cc * 16, 16)
            x0v[r, sl] = x0v[r, sl] + c * 118
            x1v[r, sl] = x1v[r, sl] + c * 4
        return carry
    lax.fori_loop(0, NB, offrow, 0)

    for b in range(NB):
        pltpu.async_copy(elem_hbm.at[x0v.at[b]], buf, sem).wait()
        pltpu.async_copy(chir_hbm.at[x1v.at[b]], buf, sem, add=True).wait()
        pltpu.sync_copy(buf, out_hbm.at[pl.ds(c * NP + s * (NB * B) + b * B, B)])


@functools.partial(
    pl.kernel,
    out_type=jax.ShapeDtypeStruct((NC, NP, DH), jnp.float32),
    mesh=_MESH,
    scratch_types=[
        pltpu.VMEM((EB // 2, B), jnp.int32),
        pltpu.VMEM((EB // 2, B), jnp.int32),
        pltpu.VMEM((EB // 2, B), jnp.int32),
        pltpu.VMEM((EB // 2, B), jnp.int32),
        [pltpu.VMEM((B, DH), jnp.float32) for _ in range(4)],
        pltpu.VMEM_SHARED((NP, DH), jnp.float32),
        [pltpu.SemaphoreType.DMA for _ in range(4)],
        [pltpu.SemaphoreType.DMA for _ in range(4)],
    ],
    compiler_params=_SC_PARAMS,
)
def _edge_agg(h_hbm, src_hbm, dst_hbm, ea0_hbm, ea1_hbm, fused_hbm, out_hbm,
              srcv, dstv, ecv, tmpv, msgs, agg_s, sem_g, sem_s):
    c = lax.axis_index("c")
    s = lax.axis_index("s")
    hb = EB // 2

    _zero_msg(msgs[0])
    for k in range(ROWS_PER_TILE // B):
        pltpu.sync_copy(msgs[0], agg_s.at[pl.ds(s * ROWS_PER_TILE + k * B, B)])

    plsc.subcore_barrier()

    def g1(b, q):
        pltpu.async_copy(fused_hbm.at[ecv.at[b]], msgs[q], sem_g[q])

    def g2(b, q):
        pltpu.async_copy(h_hbm.at[srcv.at[b]], msgs[q], sem_g[q], add=True)

    def sc(b, q):
        pltpu.async_copy(msgs[q], agg_s.at[dstv.at[b]], sem_s[q], add=True)

    def wait_g(q):
        pltpu.make_async_copy(fused_hbm.at[ecv.at[0]], msgs[q],
                              sem_g[q]).wait()

    def wait_s(q):
        pltpu.make_async_copy(msgs[q], agg_s.at[dstv.at[0]],
                              sem_s[q]).wait()

    def phase(ph, carry0):
        pltpu.sync_copy(src_hbm.at[s, pl.ds(ph * hb, hb)], srcv)
        pltpu.sync_copy(dst_hbm.at[s, pl.ds(ph * hb, hb)], dstv)
        pltpu.sync_copy(ea0_hbm.at[s, pl.ds(ph * hb, hb)], ecv)
        pltpu.sync_copy(ea1_hbm.at[s, pl.ds(ph * hb, hb)], tmpv)

        def idxrow(r, carry):
            for cc in range(B // 16):
                sl = pl.ds(cc * 16, 16)
                srcv[r, sl] = srcv[r, sl] + c * NP
                ecv[r, sl] = ecv[r, sl] * 3 + tmpv[r, sl] + c * 12
            return carry
        lax.fori_loop(0, hb, idxrow, 0)

        g1(0, 0)
        g1(1, 1)
        wait_g(0)
        g2(0, 0)

        def group(g, carry):
            for q in range(4):
                b = g * 4 + q
                q1, q2 = (q + 1) % 4, (q + 2) % 4
                wait_g(q)
                @pl.when(b >= 2)
                def _():
                    wait_s(q2)
                @pl.when(b + 2 < hb)
                def _():
                    g1(b + 2, q2)
                @pl.when(b + 1 < hb)
                def _():
                    wait_g(q1)
                    g2(b + 1, q1)

                def rrow(r, carry2):
                    for cc in range(DH // 16):
                        sl = pl.ds(cc * 16, 16)
                        msgs[q][r, sl] = jnp.maximum(msgs[q][r, sl], 0.0)
                    return carry2
                lax.fori_loop(0, B, rrow, 0)
                sc(b, q)
            return carry
        lax.fori_loop(0, hb // 4, group, 0)
        wait_s(2)
        wait_s(3)
        return carry0
    lax.fori_loop(0, 2, phase, 0)

    plsc.subcore_barrier()
    pltpu.sync_copy(agg_s.at[pl.ds(s * ROWS_PER_TILE, ROWS_PER_TILE)],
                    out_hbm.at[c, pl.ds(s * ROWS_PER_TILE, ROWS_PER_TILE)])


_GRID = 8
_RB = NP // _GRID

_HI = lax.Precision.DEFAULT
_DN = (((1,), (0,)), ((), ()))


def _mlp_body(h_ref, a_ref, w1_ref, b1_ref, w2_ref, b2_ref, z2_ref, st_ref):
    g = pl.program_id(0)
    z = jnp.concatenate([h_ref[0] + a_ref[0], h_ref[1] + a_ref[1]], axis=1)
    t = lax.dot_general(z, w1_ref[...], _DN, precision=_HI,
                        preferred_element_type=jnp.float32) + b1_ref[...]
    t = jnp.maximum(t, 0.0)
    z2 = lax.dot_general(t, w2_ref[...], _DN, precision=_HI,
                         preferred_element_type=jnp.float32) + b2_ref[...]
    rows = g * _RB + lax.broadcasted_iota(jnp.int32, (_RB, 1), 0)
    zm = jnp.where(rows < N_NODES, z2, 0.0)
    z2_ref[...] = z2
    st_ref[0, 0] = jnp.sum(zm, axis=0)
    st_ref[0, 1] = jnp.sum(zm * zm, axis=0)


def _bn_body(apply_relu, z2_ref, st_ref, g_ref, be_ref, o_ref):
    inv_n = 1.0 / N_NODES
    mu = jnp.sum(st_ref[:, 0, :], axis=0) * inv_n
    m2 = jnp.sum(st_ref[:, 1, :], axis=0) * inv_n
    var = m2 - mu * mu
    z2 = z2_ref[...]
    hn = (z2 - mu[None, :]) * lax.rsqrt(var + 1e-5)[None, :] * g_ref[...] \
        + be_ref[...]
    if apply_relu:
        hn = jnp.maximum(hn, 0.0)
    o_ref[0] = hn[:, :DH]
    o_ref[1] = hn[:, DH:]


def _mlp_bn(h2, aggs, W1, b1, W2, b2, gamma, beta, apply_relu):
    z2, st = pl.pallas_call(
        _mlp_body,
        grid=(_GRID,),
        in_specs=[
            pl.BlockSpec((NC, _RB, DH), lambda g: (0, g, 0)),
            pl.BlockSpec((NC, _RB, DH), lambda g: (0, g, 0)),
            pl.BlockSpec((D, 2 * D), lambda g: (0, 0)),
            pl.BlockSpec((1, 2 * D), lambda g: (0, 0)),
            pl.BlockSpec((2 * D, D), lambda g: (0, 0)),
            pl.BlockSpec((1, D), lambda g: (0, 0)),
        ],
        out_specs=[
            pl.BlockSpec((_RB, D), lambda g: (g, 0)),
            pl.BlockSpec((1, 2, D), lambda g: (g, 0, 0)),
        ],
        out_shape=[
            jax.ShapeDtypeStruct((NP, D), jnp.float32),
            jax.ShapeDtypeStruct((_GRID, 2, D), jnp.float32),
        ],
    )(h2, aggs, W1, b1.reshape(1, 2 * D), W2, b2.reshape(1, D))
    return pl.pallas_call(
        functools.partial(_bn_body, apply_relu),
        grid=(_GRID,),
        in_specs=[
            pl.BlockSpec((_RB, D), lambda g: (g, 0)),
            pl.BlockSpec((_GRID, 2, D), lambda g: (0, 0, 0)),
            pl.BlockSpec((1, D), lambda g: (0, 0)),
            pl.BlockSpec((1, D), lambda g: (0, 0)),
        ],
        out_specs=pl.BlockSpec((NC, _RB, DH), lambda g: (0, g, 0)),
        out_shape=jax.ShapeDtypeStruct((NC, NP, DH), jnp.float32),
    )(z2, st, gamma.reshape(1, D), beta.reshape(1, D))


def _col_split(t):
    r = t.shape[0]
    return jnp.stack([t[:, :DH], t[:, DH:]]).reshape(2 * r, DH)


def kernel(x, edge_index, edge_attr, elem_emb, chir_emb, bond_type_emb,
           bond_dir_emb, W1, b1, W2, b2, gammas, betas):
    x0 = jnp.pad(x[:, 0], (0, NP - N_NODES)).reshape(NS, NB, B)
    x1 = jnp.pad(x[:, 1], (0, NP - N_NODES)).reshape(NS, NB, B)
    src = jnp.pad(edge_index[0], (0, EP - N_EDGES)).reshape(NS, EB, B)
    dst = jnp.pad(edge_index[1], (0, EP - N_EDGES),
                  constant_values=NP - 1).reshape(NS, EB, B)
    ea0 = jnp.pad(edge_attr[:, 0], (0, EP - N_EDGES)).reshape(NS, EB, B)
    ea1 = jnp.pad(edge_attr[:, 1], (0, EP - N_EDGES)).reshape(NS, EB, B)
    fused = _col_split(
        (bond_type_emb[:, None, :] + bond_dir_emb[None, :, :]).reshape(12, D))
    elem2 = _col_split(elem_emb)
    chir2 = _col_split(chir_emb)

    h = _node_embed(x0, x1, elem2, chir2)
    for layer in range(3):
        aggs = _edge_agg(h, src, dst, ea0, ea1, fused)
        h2 = _mlp_bn(h.reshape(NC, NP, DH), aggs, W1, b1, W2, b2,
                     gammas[layer], betas[layer], apply_relu=layer < 2)
        h = h2.reshape(NC * NP, DH)
    out = h.reshape(NC, NP, DH)
    return jnp.concatenate([out[0], out[1]], axis=1)[:N_NODES]


# speedup vs baseline: 3.0834x; 2.6407x over previous
import functools

import jax
import jax.numpy as jnp
from jax import lax
from jax.experimental import pallas as pl
from jax.experimental.pallas import tpu as pltpu
from jax.experimental.pallas import tpu_sc as plsc

N_NODES = 10000
N_EDGES = 320000
D = 128
DH = 64

NC = 2
NS = 16
NW = NC * NS

B = 128
NP = 12288
EP = 327680
EB = EP // NS // B
NB = NP // NS // B
ROWS_PER_TILE = NP // NS

_MESH = plsc.VectorSubcoreMesh(core_axis_name="c", subcore_axis_name="s",
                               num_cores=NC, num_subcores=NS)
_SC_PARAMS = pltpu.CompilerParams(use_tc_tiling_on_sc=False)


def _zero_msg(buf):
    def row(r, carry):
        for cc in range(DH // 16):
            buf[r, pl.ds(cc * 16, 16)] = jnp.zeros((16,), jnp.float32)
        return carry
    lax.fori_loop(0, B, row, 0)


@functools.partial(
    pl.kernel,
    out_type=jax.ShapeDtypeStruct((NC * NP, DH), jnp.float32),
    mesh=_MESH,
    scratch_types=[
        pltpu.VMEM((NB, B), jnp.int32),
        pltpu.VMEM((NB, B), jnp.int32),
        pltpu.VMEM((B, DH), jnp.float32),
        pltpu.SemaphoreType.DMA,
    ],
    compiler_params=_SC_PARAMS,
)
def _node_embed(x0_hbm, x1_hbm, elem_hbm, chir_hbm, out_hbm, x0v, x1v, buf, sem):
    c = lax.axis_index("c")
    s = lax.axis_index("s")
    pltpu.sync_copy(x0_hbm.at[s], x0v)
    pltpu.sync_copy(x1_hbm.at[s], x1v)

    def offrow(r, carry):
        for cc in range(B // 16):
            sl = pl.ds(cc * 16, 16)
            x0v[r, sl] = x0v[r, sl] + c * 118
            x1v[r, sl] = x1v[r, sl] + c * 4
        return carry
    lax.fori_loop(0, NB, offrow, 0)

    for b in range(NB):
        pltpu.async_copy(elem_hbm.at[x0v.at[b]], buf, sem).wait()
        pltpu.async_copy(chir_hbm.at[x1v.at[b]], buf, sem, add=True).wait()
        pltpu.sync_copy(buf, out_hbm.at[pl.ds(c * NP + s * (NB * B) + b * B, B)])


@functools.partial(
    pl.kernel,
    out_type=jax.ShapeDtypeStruct((NC, NP, DH), jnp.float32),
    mesh=_MESH,
    scratch_types=[
        pltpu.VMEM((EB // 2, B), jnp.int32),
        pltpu.VMEM((EB // 2, B), jnp.int32),
        pltpu.VMEM((EB // 2, B), jnp.int32),
        pltpu.VMEM((EB // 2, B), jnp.int32),
        [pltpu.VMEM((B, DH), jnp.float32) for _ in range(4)],
        pltpu.VMEM((24, DH), jnp.float32),
        pltpu.VMEM_SHARED((NP, DH), jnp.float32),
        [pltpu.SemaphoreType.DMA for _ in range(4)],
        [pltpu.SemaphoreType.DMA for _ in range(4)],
    ],
    compiler_params=_SC_PARAMS,
)
def _edge_agg(h_hbm, src_hbm, dst_hbm, ea0_hbm, ea1_hbm, fused_hbm, out_hbm,
              srcv, dstv, ecv, tmpv, msgs, fusedv, agg_s, sem_g, sem_s):
    c = lax.axis_index("c")
    s = lax.axis_index("s")
    hb = EB // 2

    pltpu.sync_copy(fused_hbm, fusedv)

    _zero_msg(msgs[0])
    for k in range(ROWS_PER_TILE // B):
        pltpu.sync_copy(msgs[0], agg_s.at[pl.ds(s * ROWS_PER_TILE + k * B, B)])

    plsc.subcore_barrier()

    def gh(b, q):
        pltpu.async_copy(h_hbm.at[srcv.at[b]], msgs[q], sem_g[q])

    def sc(b, q):
        pltpu.async_copy(msgs[q], agg_s.at[dstv.at[b]], sem_s[q], add=True)

    def wait_g(q):
        pltpu.make_async_copy(h_hbm.at[srcv.at[0]], msgs[q],
                              sem_g[q]).wait()

    def wait_s(q):
        pltpu.make_async_copy(msgs[q], agg_s.at[dstv.at[0]],
                              sem_s[q]).wait()

    def phase(ph, carry0):
        pltpu.sync_copy(src_hbm.at[s, pl.ds(ph * hb, hb)], srcv)
        pltpu.sync_copy(dst_hbm.at[s, pl.ds(ph * hb, hb)], dstv)
        pltpu.sync_copy(ea0_hbm.at[s, pl.ds(ph * hb, hb)], ecv)
        pltpu.sync_copy(ea1_hbm.at[s, pl.ds(ph * hb, hb)], tmpv)

        def idxrow(r, carry):
            for cc in range(B // 16):
                sl = pl.ds(cc * 16, 16)
                srcv[r, sl] = srcv[r, sl] + c * NP
                ecv[r, sl] = ecv[r, sl] * 3 + tmpv[r, sl] + c * 12
            return carry
        lax.fori_loop(0, hb, idxrow, 0)

        gh(0, 0)
        gh(1, 1)

        def group(g, carry):
            for q in range(4):
                b = g * 4 + q
                q2 = (q + 2) % 4
                wait_g(q)
                @pl.when(b >= 2)
                def _():
                    wait_s(q2)
                @pl.when(b + 2 < hb)
                def _():
                    gh(b + 2, q2)

                def rgroup(rg, carry2):
                    ec16 = ecv[b, pl.ds(rg * 16, 16)]
                    for i in range(16):
                        er = ec16[i]
                        r = rg * 16 + i
                        for cc in range(DH // 16):
                            sl = pl.ds(cc * 16, 16)
                            msgs[q][r, sl] = jnp.maximum(
                                msgs[q][r, sl] + fusedv[er, sl], 0.0)
                    return carry2
                lax.fori_loop(0, B // 16, rgroup, 0)
                sc(b, q)
            return carry
        lax.fori_loop(0, hb // 4, group, 0)
        wait_s(2)
        wait_s(3)
        return carry0
    lax.fori_loop(0, 2, phase, 0)

    plsc.subcore_barrier()
    pltpu.sync_copy(agg_s.at[pl.ds(s * ROWS_PER_TILE, ROWS_PER_TILE)],
                    out_hbm.at[c, pl.ds(s * ROWS_PER_TILE, ROWS_PER_TILE)])


_GRID = 8
_RB = NP // _GRID

_HI = lax.Precision.DEFAULT
_DN = (((1,), (0,)), ((), ()))


def _mlp_body(h_ref, a_ref, w1_ref, b1_ref, w2_ref, b2_ref, z2_ref, st_ref):
    g = pl.program_id(0)
    z = jnp.concatenate([h_ref[0] + a_ref[0], h_ref[1] + a_ref[1]], axis=1)
    t = lax.dot_general(z, w1_ref[...], _DN, precision=_HI,
                        preferred_element_type=jnp.float32) + b1_ref[...]
    t = jnp.maximum(t, 0.0)
    z2 = lax.dot_general(t, w2_ref[...], _DN, precision=_HI,
                         preferred_element_type=jnp.float32) + b2_ref[...]
    rows = g * _RB + lax.broadcasted_iota(jnp.int32, (_RB, 1), 0)
    zm = jnp.where(rows < N_NODES, z2, 0.0)
    z2_ref[...] = z2
    st_ref[0, 0] = jnp.sum(zm, axis=0)
    st_ref[0, 1] = jnp.sum(zm * zm, axis=0)


def _bn_body(apply_relu, z2_ref, st_ref, g_ref, be_ref, o_ref):
    inv_n = 1.0 / N_NODES
    mu = jnp.sum(st_ref[:, 0, :], axis=0) * inv_n
    m2 = jnp.sum(st_ref[:, 1, :], axis=0) * inv_n
    var = m2 - mu * mu
    z2 = z2_ref[...]
    hn = (z2 - mu[None, :]) * lax.rsqrt(var + 1e-5)[None, :] * g_ref[...] \
        + be_ref[...]
    if apply_relu:
        hn = jnp.maximum(hn, 0.0)
    o_ref[0] = hn[:, :DH]
    o_ref[1] = hn[:, DH:]


def _mlp_bn(h2, aggs, W1, b1, W2, b2, gamma, beta, apply_relu):
    z2, st = pl.pallas_call(
        _mlp_body,
        grid=(_GRID,),
        in_specs=[
            pl.BlockSpec((NC, _RB, DH), lambda g: (0, g, 0)),
            pl.BlockSpec((NC, _RB, DH), lambda g: (0, g, 0)),
            pl.BlockSpec((D, 2 * D), lambda g: (0, 0)),
            pl.BlockSpec((1, 2 * D), lambda g: (0, 0)),
            pl.BlockSpec((2 * D, D), lambda g: (0, 0)),
            pl.BlockSpec((1, D), lambda g: (0, 0)),
        ],
        out_specs=[
            pl.BlockSpec((_RB, D), lambda g: (g, 0)),
            pl.BlockSpec((1, 2, D), lambda g: (g, 0, 0)),
        ],
        out_shape=[
            jax.ShapeDtypeStruct((NP, D), jnp.float32),
            jax.ShapeDtypeStruct((_GRID, 2, D), jnp.float32),
        ],
    )(h2, aggs, W1, b1.reshape(1, 2 * D), W2, b2.reshape(1, D))
    return pl.pallas_call(
        functools.partial(_bn_body, apply_relu),
        grid=(_GRID,),
        in_specs=[
            pl.BlockSpec((_RB, D), lambda g: (g, 0)),
            pl.BlockSpec((_GRID, 2, D), lambda g: (0, 0, 0)),
            pl.BlockSpec((1, D), lambda g: (0, 0)),
            pl.BlockSpec((1, D), lambda g: (0, 0)),
        ],
        out_specs=pl.BlockSpec((NC, _RB, DH), lambda g: (0, g, 0)),
        out_shape=jax.ShapeDtypeStruct((NC, NP, DH), jnp.float32),
    )(z2, st, gamma.reshape(1, D), beta.reshape(1, D))


def _col_split(t):
    r = t.shape[0]
    return jnp.stack([t[:, :DH], t[:, DH:]]).reshape(2 * r, DH)


def kernel(x, edge_index, edge_attr, elem_emb, chir_emb, bond_type_emb,
           bond_dir_emb, W1, b1, W2, b2, gammas, betas):
    x0 = jnp.pad(x[:, 0], (0, NP - N_NODES)).reshape(NS, NB, B)
    x1 = jnp.pad(x[:, 1], (0, NP - N_NODES)).reshape(NS, NB, B)
    src = jnp.pad(edge_index[0], (0, EP - N_EDGES)).reshape(NS, EB, B)
    dst = jnp.pad(edge_index[1], (0, EP - N_EDGES),
                  constant_values=NP - 1).reshape(NS, EB, B)
    ea0 = jnp.pad(edge_attr[:, 0], (0, EP - N_EDGES)).reshape(NS, EB, B)
    ea1 = jnp.pad(edge_attr[:, 1], (0, EP - N_EDGES)).reshape(NS, EB, B)
    fused = _col_split(
        (bond_type_emb[:, None, :] + bond_dir_emb[None, :, :]).reshape(12, D))
    elem2 = _col_split(elem_emb)
    chir2 = _col_split(chir_emb)

    h = _node_embed(x0, x1, elem2, chir2)
    for layer in range(3):
        aggs = _edge_agg(h, src, dst, ea0, ea1, fused)
        h2 = _mlp_bn(h.reshape(NC, NP, DH), aggs, W1, b1, W2, b2,
                     gammas[layer], betas[layer], apply_relu=layer < 2)
        h = h2.reshape(NC * NP, DH)
    out = h.reshape(NC, NP, DH)
    return jnp.concatenate([out[0], out[1]], axis=1)[:N_NODES]


# speedup vs baseline: 3.2849x; 1.0653x over previous
import functools

import jax
import jax.numpy as jnp
from jax import lax
from jax.experimental import pallas as pl
from jax.experimental.pallas import tpu as pltpu
from jax.experimental.pallas import tpu_sc as plsc

N_NODES = 10000
N_EDGES = 320000
D = 128
DH = 64

NC = 2
NS = 16
NW = NC * NS

B = 128
NP = 12288
EP = 327680
EB = EP // NS // B
NB = NP // NS // B
ROWS_PER_TILE = NP // NS

_MESH = plsc.VectorSubcoreMesh(core_axis_name="c", subcore_axis_name="s",
                               num_cores=NC, num_subcores=NS)
_SC_PARAMS = pltpu.CompilerParams(use_tc_tiling_on_sc=False)


def _zero_msg(buf):
    def row(r, carry):
        for cc in range(DH // 16):
            buf[r, pl.ds(cc * 16, 16)] = jnp.zeros((16,), jnp.float32)
        return carry
    lax.fori_loop(0, B, row, 0)


@functools.partial(
    pl.kernel,
    out_type=jax.ShapeDtypeStruct((NC * NP, DH), jnp.float32),
    mesh=_MESH,
    scratch_types=[
        pltpu.VMEM((NB, B), jnp.int32),
        pltpu.VMEM((NB, B), jnp.int32),
        [pltpu.VMEM((B, DH), jnp.float32) for _ in range(2)],
        pltpu.VMEM((8, DH), jnp.float32),
        [pltpu.SemaphoreType.DMA for _ in range(2)],
    ],
    compiler_params=_SC_PARAMS,
)
def _node_embed(x0_hbm, x1_hbm, elem_hbm, chir_hbm, out_hbm, x0v, x1v, bufs,
                chirv, sems):
    c = lax.axis_index("c")
    s = lax.axis_index("s")
    pltpu.sync_copy(x0_hbm.at[s], x0v)
    pltpu.sync_copy(x1_hbm.at[s], x1v)
    pltpu.sync_copy(chir_hbm, chirv)

    def offrow(r, carry):
        for cc in range(B // 16):
            sl = pl.ds(cc * 16, 16)
            x0v[r, sl] = x0v[r, sl] + c * 118
        return carry
    lax.fori_loop(0, NB, offrow, 0)

    for b in range(2):
        pltpu.async_copy(elem_hbm.at[x0v.at[b]], bufs[b], sems[b])
    for b in range(NB):
        q = b % 2
        pltpu.make_async_copy(elem_hbm.at[x0v.at[b]], bufs[q], sems[q]).wait()

        def rgroup(rg, carry, b=b, q=q):
            ec16 = x1v[b, pl.ds(rg * 16, 16)]
            for i in range(16):
                er = ec16[i] + c * 4
                r = rg * 16 + i
                for cc in range(DH // 16):
                    sl = pl.ds(cc * 16, 16)
                    bufs[q][r, sl] = bufs[q][r, sl] + chirv[er, sl]
            return carry
        lax.fori_loop(0, B // 16, rgroup, 0)
        pltpu.sync_copy(bufs[q],
                        out_hbm.at[pl.ds(c * NP + s * (NB * B) + b * B, B)])
        if b + 2 < NB:
            pltpu.async_copy(elem_hbm.at[x0v.at[b + 2]], bufs[q], sems[q])


@functools.partial(
    pl.kernel,
    out_type=jax.ShapeDtypeStruct((NC, NP, DH), jnp.float32),
    mesh=_MESH,
    scratch_types=[
        pltpu.VMEM((EB // 2, B), jnp.int32),
        pltpu.VMEM((EB // 2, B), jnp.int32),
        pltpu.VMEM((EB // 2, B), jnp.int32),
        pltpu.VMEM((EB // 2, B), jnp.int32),
        [pltpu.VMEM((B, DH), jnp.float32) for _ in range(4)],
        pltpu.VMEM((24, DH), jnp.float32),
        pltpu.VMEM_SHARED((NP, DH), jnp.float32),
        [pltpu.SemaphoreType.DMA for _ in range(4)],
        [pltpu.SemaphoreType.DMA for _ in range(4)],
    ],
    compiler_params=_SC_PARAMS,
)
def _edge_agg(h_hbm, src_hbm, dst_hbm, ea0_hbm, ea1_hbm, fused_hbm, out_hbm,
              srcv, dstv, ecv, tmpv, msgs, fusedv, agg_s, sem_g, sem_s):
    c = lax.axis_index("c")
    s = lax.axis_index("s")
    hb = EB // 2

    pltpu.sync_copy(fused_hbm, fusedv)

    _zero_msg(msgs[0])
    for k in range(ROWS_PER_TILE // B):
        pltpu.sync_copy(msgs[0], agg_s.at[pl.ds(s * ROWS_PER_TILE + k * B, B)])

    plsc.subcore_barrier()

    def gh(b, q):
        pltpu.async_copy(h_hbm.at[srcv.at[b]], msgs[q], sem_g[q])

    def sc(b, q):
        pltpu.async_copy(msgs[q], agg_s.at[dstv.at[b]], sem_s[q], add=True)

    def wait_g(q):
        pltpu.make_async_copy(h_hbm.at[srcv.at[0]], msgs[q],
                              sem_g[q]).wait()

    def wait_s(q):
        pltpu.make_async_copy(msgs[q], agg_s.at[dstv.at[0]],
                              sem_s[q]).wait()

    def phase(ph, carry0):
        pltpu.sync_copy(src_hbm.at[s, pl.ds(ph * hb, hb)], srcv)
        pltpu.sync_copy(dst_hbm.at[s, pl.ds(ph * hb, hb)], dstv)
        pltpu.sync_copy(ea0_hbm.at[s, pl.ds(ph * hb, hb)], ecv)
        pltpu.sync_copy(ea1_hbm.at[s, pl.ds(ph * hb, hb)], tmpv)

        def idxrow(r, carry):
            for cc in range(B // 16):
                sl = pl.ds(cc * 16, 16)
                srcv[r, sl] = srcv[r, sl] + c * NP
                ecv[r, sl] = ecv[r, sl] * 3 + tmpv[r, sl] + c * 12
            return carry
        lax.fori_loop(0, hb, idxrow, 0)

        gh(0, 0)
        gh(1, 1)

        def group(g, carry):
            for q in range(4):
                b = g * 4 + q
                q2 = (q + 2) % 4
                wait_g(q)
                @pl.when(b >= 2)
                def _():
                    wait_s(q2)
                @pl.when(b + 2 < hb)
                def _():
                    gh(b + 2, q2)

                def rgroup(rg, carry2):
                    ec16 = ecv[b, pl.ds(rg * 16, 16)]
                    for i in range(16):
                        er = ec16[i]
                        r = rg * 16 + i
                        for cc in range(DH // 16):
                            sl = pl.ds(cc * 16, 16)
                            msgs[q][r, sl] = jnp.maximum(
                                msgs[q][r, sl] + fusedv[er, sl], 0.0)
                    return carry2
                lax.fori_loop(0, B // 16, rgroup, 0)
                sc(b, q)
            return carry
        lax.fori_loop(0, hb // 4, group, 0)
        wait_s(2)
        wait_s(3)
        return carry0
    lax.fori_loop(0, 2, phase, 0)

    plsc.subcore_barrier()
    pltpu.sync_copy(agg_s.at[pl.ds(s * ROWS_PER_TILE, ROWS_PER_TILE)],
                    out_hbm.at[c, pl.ds(s * ROWS_PER_TILE, ROWS_PER_TILE)])


_GRID = 8
_RB = NP // _GRID

_HI = lax.Precision.DEFAULT
_DN = (((1,), (0,)), ((), ()))


def _mlp_body(h_ref, a_ref, w1_ref, b1_ref, w2_ref, b2_ref, z2_ref, st_ref):
    g = pl.program_id(0)
    z = jnp.concatenate([h_ref[0] + a_ref[0], h_ref[1] + a_ref[1]], axis=1)
    t = lax.dot_general(z, w1_ref[...], _DN, precision=_HI,
                        preferred_element_type=jnp.float32) + b1_ref[...]
    t = jnp.maximum(t, 0.0)
    z2 = lax.dot_general(t, w2_ref[...], _DN, precision=_HI,
                         preferred_element_type=jnp.float32) + b2_ref[...]
    rows = g * _RB + lax.broadcasted_iota(jnp.int32, (_RB, 1), 0)
    zm = jnp.where(rows < N_NODES, z2, 0.0)
    z2_ref[...] = z2
    st_ref[0, 0] = jnp.sum(zm, axis=0)
    st_ref[0, 1] = jnp.sum(zm * zm, axis=0)


def _bn_body(apply_relu, z2_ref, st_ref, g_ref, be_ref, o_ref):
    inv_n = 1.0 / N_NODES
    mu = jnp.sum(st_ref[:, 0, :], axis=0) * inv_n
    m2 = jnp.sum(st_ref[:, 1, :], axis=0) * inv_n
    var = m2 - mu * mu
    z2 = z2_ref[...]
    hn = (z2 - mu[None, :]) * lax.rsqrt(var + 1e-5)[None, :] * g_ref[...] \
        + be_ref[...]
    if apply_relu:
        hn = jnp.maximum(hn, 0.0)
    o_ref[0] = hn[:, :DH]
    o_ref[1] = hn[:, DH:]


def _mlp_bn(h2, aggs, W1, b1, W2, b2, gamma, beta, apply_relu):
    z2, st = pl.pallas_call(
        _mlp_body,
        grid=(_GRID,),
        in_specs=[
            pl.BlockSpec((NC, _RB, DH), lambda g: (0, g, 0)),
            pl.BlockSpec((NC, _RB, DH), lambda g: (0, g, 0)),
            pl.BlockSpec((D, 2 * D), lambda g: (0, 0)),
            pl.BlockSpec((1, 2 * D), lambda g: (0, 0)),
            pl.BlockSpec((2 * D, D), lambda g: (0, 0)),
            pl.BlockSpec((1, D), lambda g: (0, 0)),
        ],
        out_specs=[
            pl.BlockSpec((_RB, D), lambda g: (g, 0)),
            pl.BlockSpec((1, 2, D), lambda g: (g, 0, 0)),
        ],
        out_shape=[
            jax.ShapeDtypeStruct((NP, D), jnp.float32),
            jax.ShapeDtypeStruct((_GRID, 2, D), jnp.float32),
        ],
    )(h2, aggs, W1, b1.reshape(1, 2 * D), W2, b2.reshape(1, D))
    return pl.pallas_call(
        functools.partial(_bn_body, apply_relu),
        grid=(_GRID,),
        in_specs=[
            pl.BlockSpec((_RB, D), lambda g: (g, 0)),
            pl.BlockSpec((_GRID, 2, D), lambda g: (0, 0, 0)),
            pl.BlockSpec((1, D), lambda g: (0, 0)),
            pl.BlockSpec((1, D), lambda g: (0, 0)),
        ],
        out_specs=pl.BlockSpec((NC, _RB, DH), lambda g: (0, g, 0)),
        out_shape=jax.ShapeDtypeStruct((NC, NP, DH), jnp.float32),
    )(z2, st, gamma.reshape(1, D), beta.reshape(1, D))


def _col_split(t):
    r = t.shape[0]
    return jnp.stack([t[:, :DH], t[:, DH:]]).reshape(2 * r, DH)


def kernel(x, edge_index, edge_attr, elem_emb, chir_emb, bond_type_emb,
           bond_dir_emb, W1, b1, W2, b2, gammas, betas):
    x0 = jnp.pad(x[:, 0], (0, NP - N_NODES)).reshape(NS, NB, B)
    x1 = jnp.pad(x[:, 1], (0, NP - N_NODES)).reshape(NS, NB, B)
    src = jnp.pad(edge_index[0], (0, EP - N_EDGES)).reshape(NS, EB, B)
    dst = jnp.pad(edge_index[1], (0, EP - N_EDGES),
                  constant_values=NP - 1).reshape(NS, EB, B)
    ea0 = jnp.pad(edge_attr[:, 0], (0, EP - N_EDGES)).reshape(NS, EB, B)
    ea1 = jnp.pad(edge_attr[:, 1], (0, EP - N_EDGES)).reshape(NS, EB, B)
    fused = _col_split(
        (bond_type_emb[:, None, :] + bond_dir_emb[None, :, :]).reshape(12, D))
    elem2 = _col_split(elem_emb)
    chir2 = _col_split(chir_emb)

    h = _node_embed(x0, x1, elem2, chir2)
    for layer in range(3):
        aggs = _edge_agg(h, src, dst, ea0, ea1, fused)
        h2 = _mlp_bn(h.reshape(NC, NP, DH), aggs, W1, b1, W2, b2,
                     gammas[layer], betas[layer], apply_relu=layer < 2)
        h = h2.reshape(NC * NP, DH)
    out = h.reshape(NC, NP, DH)
    return jnp.concatenate([out[0], out[1]], axis=1)[:N_NODES]
